# Initial kernel scaffold; baseline (speedup 1.0000x reference)
import jax
import jax.numpy as jnp
from jax.experimental import pallas as pl


def kernel(x, b):
    raise NotImplementedError("write your pallas kernel here")


# speedup vs baseline: 11.0804x; 11.0804x over previous
import functools

import jax
import jax.numpy as jnp
from jax import lax
from jax.experimental import pallas as pl
from jax.experimental.pallas import tpu as pltpu
from jax.experimental.pallas import tpu_sc as plsc

L = 16
NC = 2
NS = 16
NW = NC * NS

R = 128
N = 32768
RPW = R // NW
NCHUNK = N // L
NEG = -1e30


def _vdiv_scalar(a, b):
  av = jnp.broadcast_to(a, (L,))
  bv = jnp.broadcast_to(b, (L,))
  return jnp.max(av / bv)


def _sc_body(x_hbm, b_hbm, out_hbm, buf, cand, bstage):
  wid = lax.axis_index("s") * NC + lax.axis_index("c")

  pltpu.sync_copy(b_hbm, bstage)
  bv = bstage[...]
  nbb = -1.0 / (1.0 + jnp.exp(-bv))

  def do_row(j, carry):
    base = (wid * RPW + j) * N
    pltpu.sync_copy(x_hbm.at[pl.ds(base, N)], buf)

    def max_body(i, acc):
      return jnp.maximum(acc, buf[pl.ds(i * L, L)])

    acc = lax.fori_loop(0, NCHUNK, max_body,
                        jnp.full((L,), NEG, jnp.float32), unroll=4)
    m = jnp.max(acc)
    thr = m - 1.0

    def comp_body(i, cnt):
      v = buf[pl.ds(i * L, L)]
      msk = v > thr
      plsc.store_compressed(cand.at[pl.ds(cnt, L)], v, mask=msk)
      return cnt + jnp.max(plsc.all_reduce_population_count(msk))

    ncand = lax.fori_loop(0, NCHUNK, comp_body, jnp.int32(0), unroll=4)
    cand[pl.ds(ncand, L)] = jnp.full((L,), NEG, jnp.float32)
    nch = lax.shift_right_logical(ncand + (L - 1), 4)

    def mich_cond(c):
      it, tau_prev, tau = c
      return (tau > tau_prev) & (it < 64)

    def mich_body(c):
      it, tau_prev, tau = c

      def sum_body(i, sk):
        s, k = sk
        v = cand[pl.ds(i * L, L)]
        msk = v > tau
        return (s + jnp.where(msk, v, 0.0),
                k + jnp.where(msk, 1.0, 0.0))

      s, k = lax.fori_loop(
          0, nch, sum_body,
          (jnp.zeros((L,), jnp.float32), jnp.zeros((L,), jnp.float32)))
      tau_new = _vdiv_scalar(jnp.sum(s) - 1.0, jnp.sum(k))
      return (it + 1, tau, tau_new)

    _, _, tau = lax.while_loop(mich_cond, mich_body,
                               (jnp.int32(0), thr - 1.0, thr))

    def out_body(i, _):
      v = buf[pl.ds(i * L, L)]
      p = jnp.maximum(v - tau, 0.0)
      buf[pl.ds(i * L, L)] = jnp.exp(nbb * p)
      return 0

    lax.fori_loop(0, NCHUNK, out_body, 0, unroll=4)
    pltpu.sync_copy(buf, out_hbm.at[pl.ds(base, N)])
    return carry

  lax.fori_loop(0, RPW, do_row, 0)


def kernel(x, b):
  bvec = jnp.full((L,), b, dtype=jnp.float32)
  x1 = x.reshape(R * N)
  mesh = plsc.VectorSubcoreMesh(core_axis_name="c", subcore_axis_name="s")
  out = pl.kernel(
      _sc_body,
      out_type=jax.ShapeDtypeStruct((R * N,), jnp.float32),
      mesh=mesh,
      compiler_params=pltpu.CompilerParams(needs_layout_passes=False),
      scratch_types=[
          pltpu.VMEM((N,), jnp.float32),
          pltpu.VMEM((N + L,), jnp.float32),
          pltpu.VMEM((L,), jnp.float32),
      ],
  )(x1, bvec)
  return out.reshape(R, N)


# speedup vs baseline: 12.1118x; 1.0931x over previous
import functools

import jax
import jax.numpy as jnp
from jax import lax
from jax.experimental import pallas as pl
from jax.experimental.pallas import tpu as pltpu
from jax.experimental.pallas import tpu_sc as plsc

L = 16
NC = 2
NS = 16
NW = NC * NS

R = 128
N = 32768
RPW = R // NW
NCHUNK = N // L
NEG = -1e30


def _vdiv_scalar(a, b):
  av = jnp.broadcast_to(a, (L,))
  bv = jnp.broadcast_to(b, (L,))
  return jnp.max(av / bv)


def _sc_body(x_hbm, b_hbm, out_hbm, buf, cand, bstage):
  wid = lax.axis_index("s") * NC + lax.axis_index("c")

  pltpu.sync_copy(b_hbm, bstage)
  bv = bstage[...]
  nbb = -1.0 / (1.0 + jnp.exp(-bv))

  def do_row(j, carry):
    base = (wid * RPW + j) * N
    pltpu.sync_copy(x_hbm.at[pl.ds(base, N)], buf)

    def max_body(i, acc):
      return jnp.maximum(acc, buf[pl.ds(i * L, L)])

    acc = lax.fori_loop(0, NCHUNK, max_body,
                        jnp.full((L,), NEG, jnp.float32), unroll=16)
    m = jnp.max(acc)
    thr = m - 1.0

    def comp_body(i, cnt):
      v = buf[pl.ds(i * L, L)]
      msk = v > thr
      plsc.store_compressed(cand.at[pl.ds(cnt, L)], v, mask=msk)
      return cnt + jnp.max(plsc.all_reduce_population_count(msk))

    ncand = lax.fori_loop(0, NCHUNK, comp_body, jnp.int32(0), unroll=8)
    cand[pl.ds(ncand, L)] = jnp.full((L,), NEG, jnp.float32)
    nch = lax.shift_right_logical(ncand + (L - 1), 4)

    def mich_cond(c):
      it, tau_prev, tau = c
      return (tau > tau_prev) & (it < 64)

    def mich_body(c):
      it, tau_prev, tau = c

      def sum_body(i, sk):
        s, k = sk
        v = cand[pl.ds(i * L, L)]
        msk = v > tau
        return (s + jnp.where(msk, v, 0.0),
                k + jnp.where(msk, 1.0, 0.0))

      s, k = lax.fori_loop(
          0, nch, sum_body,
          (jnp.zeros((L,), jnp.float32), jnp.zeros((L,), jnp.float32)))
      tau_new = _vdiv_scalar(jnp.sum(s) - 1.0, jnp.sum(k))
      return (it + 1, tau, tau_new)

    _, _, tau = lax.while_loop(mich_cond, mich_body,
                               (jnp.int32(0), thr - 1.0, thr))

    def out_body(i, _):
      v = buf[pl.ds(i * L, L)]
      p = jnp.maximum(v - tau, 0.0)
      buf[pl.ds(i * L, L)] = jnp.exp(nbb * p)
      return 0

    lax.fori_loop(0, NCHUNK, out_body, 0, unroll=8)
    pltpu.sync_copy(buf, out_hbm.at[pl.ds(base, N)])
    return carry

  lax.fori_loop(0, RPW, do_row, 0)


def kernel(x, b):
  bvec = jnp.full((L,), b, dtype=jnp.float32)
  x1 = x.reshape(R * N)
  mesh = plsc.VectorSubcoreMesh(core_axis_name="c", subcore_axis_name="s")
  out = pl.kernel(
      _sc_body,
      out_type=jax.ShapeDtypeStruct((R * N,), jnp.float32),
      mesh=mesh,
      compiler_params=pltpu.CompilerParams(needs_layout_passes=False),
      scratch_types=[
          pltpu.VMEM((N,), jnp.float32),
          pltpu.VMEM((N + L,), jnp.float32),
          pltpu.VMEM((L,), jnp.float32),
      ],
  )(x1, bvec)
  return out.reshape(R, N)
